# baseline (speedup 1.0000x reference)
import functools
import jax
import jax.numpy as jnp
from jax import lax
from jax.experimental import pallas as pl
from jax.experimental.pallas import tpu as pltpu
from jax.experimental.pallas import tpu_sc as plsc

_B, _C, _N = 8, 3, 16384
_M = 512
_L = 16
_CHUNKS = _N // _L
_UNROLL = 4


def _lane_permute(v, perm):
    dnums = lax.GatherDimensionNumbers(
        offset_dims=(), collapsed_slice_dims=(0,), start_index_map=(0,))
    return lax.gather(v, perm[:, None], dnums, (1,),
                      mode=lax.GatherScatterMode.PROMISE_IN_BOUNDS)


def _splat_at(ref, pos):
    return lax.broadcast(ref[pl.ds(pos, _L)][0], (_L,))


def _fps_body(pts_hbm, out_hbm, x_v, y_v, z_v, dists_v, idx_v):
    cid = lax.axis_index("c")
    sid = lax.axis_index("s")
    wid = sid * 2 + cid

    @pl.when(wid < _B)
    def _body():
        base_off = wid * (_C * _N)
        pltpu.sync_copy(pts_hbm.at[pl.ds(base_off, _N)], x_v.at[pl.ds(0, _N)])
        pltpu.sync_copy(pts_hbm.at[pl.ds(base_off + _N, _N)],
                        y_v.at[pl.ds(0, _N)])
        pltpu.sync_copy(pts_hbm.at[pl.ds(base_off + 2 * _N, _N)],
                        z_v.at[pl.ds(0, _N)])

        lanes = lax.iota(jnp.int32, _L)
        zeros = jnp.zeros((_L,), jnp.int32)
        intmax = jnp.full((_L,), 2147483647, jnp.int32)
        inf16 = jnp.full((_L,), jnp.inf, jnp.float32)

        def _init(j, carry):
            dists_v[pl.ds(j * _L, _L)] = inf16
            return carry

        lax.fori_loop(0, _CHUNKS, _init, 0)

        qx = _splat_at(x_v, 0)
        qy = _splat_at(y_v, 0)
        qz = _splat_at(z_v, 0)

        def _outer(i, carry):
            qx, qy, qz, pending = carry

            def _chunk(j, st):
                runmax, runidx = st
                jbase = j * (_L * _UNROLL)
                for u in range(_UNROLL):
                    base = jbase + u * _L
                    dx = x_v[pl.ds(base, _L)] - qx
                    dy = y_v[pl.ds(base, _L)] - qy
                    dz = z_v[pl.ds(base, _L)] - qz
                    d = dx * dx + dy * dy
                    d = d + dz * dz
                    nd = jnp.minimum(dists_v[pl.ds(base, _L)], d)
                    dists_v[pl.ds(base, _L)] = nd
                    gt = nd > runmax
                    runmax = jnp.where(gt, nd, runmax)
                    runidx = jnp.where(gt, lanes + base, runidx)
                return runmax, runidx

            runmax, runidx = lax.fori_loop(
                0, _CHUNKS // _UNROLL, _chunk,
                (jnp.full((_L,), -jnp.inf, jnp.float32), zeros))

            for shift in (8, 4, 2, 1):
                perm = (lanes + shift) & (_L - 1)
                bv = _lane_permute(runmax, perm)
                bi = _lane_permute(runidx, perm)
                take = (bv > runmax) | ((bv == runmax) & (bi < runidx))
                runmax = jnp.where(take, bv, runmax)
                runidx = jnp.where(take, bi, runidx)
            nxt_s = runidx[0]

            pending = jnp.where(lanes == (i & (_L - 1)), runidx, pending)

            @pl.when((i & (_L - 1)) == _L - 1)
            def _flush():
                idx_v[pl.ds(i - (_L - 1), _L)] = pending

            qx = _splat_at(x_v, nxt_s)
            qy = _splat_at(y_v, nxt_s)
            qz = _splat_at(z_v, nxt_s)
            return qx, qy, qz, pending

        lax.fori_loop(1, _M, _outer, (qx, qy, qz, zeros))
        pltpu.sync_copy(idx_v, out_hbm.at[wid])


@functools.partial(
    pl.kernel,
    mesh=plsc.VectorSubcoreMesh(core_axis_name="c", subcore_axis_name="s"),
    out_type=jax.ShapeDtypeStruct((_B, _M), jnp.int32),
    scratch_types=[
        pltpu.VMEM((_N + _L,), jnp.float32),
        pltpu.VMEM((_N + _L,), jnp.float32),
        pltpu.VMEM((_N + _L,), jnp.float32),
        pltpu.VMEM((_N,), jnp.float32),
        pltpu.VMEM((_M,), jnp.int32),
    ],
)
def _fps(pts_hbm, out_hbm, x_v, y_v, z_v, dists_v, idx_v):
    _fps_body(pts_hbm, out_hbm, x_v, y_v, z_v, dists_v, idx_v)


def kernel(pt_coordinates):
    return _fps(pt_coordinates.reshape(-1))


# speedup vs baseline: 1.0148x; 1.0148x over previous
import functools
import jax
import jax.numpy as jnp
from jax import lax
from jax.experimental import pallas as pl
from jax.experimental.pallas import tpu as pltpu
from jax.experimental.pallas import tpu_sc as plsc

_B, _C, _N = 8, 3, 16384
_M = 512
_L = 16
_CHUNKS = _N // _L
_UNROLL = 4


def _lane_permute(v, perm):
    dnums = lax.GatherDimensionNumbers(
        offset_dims=(), collapsed_slice_dims=(0,), start_index_map=(0,))
    return lax.gather(v, perm[:, None], dnums, (1,),
                      mode=lax.GatherScatterMode.PROMISE_IN_BOUNDS)


def _splat_at(ref, pos):
    return lax.broadcast(ref[pl.ds(pos, _L)][0], (_L,))


def _fps_body(pts_hbm, out_hbm, x_v, y_v, z_v, dists_v, idx_v):
    cid = lax.axis_index("c")
    sid = lax.axis_index("s")
    wid = sid * 2 + cid

    @pl.when(wid < _B)
    def _body():
        base_off = wid * (_C * _N)
        pltpu.sync_copy(pts_hbm.at[pl.ds(base_off, _N)], x_v.at[pl.ds(0, _N)])
        pltpu.sync_copy(pts_hbm.at[pl.ds(base_off + _N, _N)],
                        y_v.at[pl.ds(0, _N)])
        pltpu.sync_copy(pts_hbm.at[pl.ds(base_off + 2 * _N, _N)],
                        z_v.at[pl.ds(0, _N)])

        lanes = lax.iota(jnp.int32, _L)
        zeros = jnp.zeros((_L,), jnp.int32)
        intmax = jnp.full((_L,), 2147483647, jnp.int32)
        inf16 = jnp.full((_L,), jnp.inf, jnp.float32)

        def _init(j, carry):
            dists_v[pl.ds(j * _L, _L)] = inf16
            return carry

        lax.fori_loop(0, _CHUNKS, _init, 0)

        qx = _splat_at(x_v, 0)
        qy = _splat_at(y_v, 0)
        qz = _splat_at(z_v, 0)

        def _outer(i, carry):
            qx, qy, qz, pending = carry

            neg16 = jnp.full((_L,), -jnp.inf, jnp.float32)
            acc0 = (neg16, zeros) * _UNROLL

            @plsc.parallel_loop(0, _CHUNKS // _UNROLL, 1, unroll=2,
                                carry=acc0)
            def _chunk(j, st):
                jbase = j * (_L * _UNROLL)
                out = []
                for u in range(_UNROLL):
                    runmax, runidx = st[2 * u], st[2 * u + 1]
                    base = jbase + u * _L
                    dx = x_v[pl.ds(base, _L)] - qx
                    dy = y_v[pl.ds(base, _L)] - qy
                    dz = z_v[pl.ds(base, _L)] - qz
                    d = dx * dx + dy * dy
                    d = d + dz * dz
                    nd = jnp.minimum(dists_v[pl.ds(base, _L)], d)
                    dists_v[pl.ds(base, _L)] = nd
                    gt = nd > runmax
                    out.append(jnp.where(gt, nd, runmax))
                    out.append(jnp.where(gt, lanes + base, runidx))
                return tuple(out)

            acc = _chunk
            runmax, runidx = acc[0], acc[1]
            for u in range(1, _UNROLL):
                bv, bi = acc[2 * u], acc[2 * u + 1]
                take = (bv > runmax) | ((bv == runmax) & (bi < runidx))
                runmax = jnp.where(take, bv, runmax)
                runidx = jnp.where(take, bi, runidx)

            for shift in (8, 4, 2, 1):
                perm = (lanes + shift) & (_L - 1)
                bv = _lane_permute(runmax, perm)
                bi = _lane_permute(runidx, perm)
                take = (bv > runmax) | ((bv == runmax) & (bi < runidx))
                runmax = jnp.where(take, bv, runmax)
                runidx = jnp.where(take, bi, runidx)
            nxt_s = runidx[0]

            pending = jnp.where(lanes == (i & (_L - 1)), runidx, pending)

            @pl.when((i & (_L - 1)) == _L - 1)
            def _flush():
                idx_v[pl.ds(i - (_L - 1), _L)] = pending

            qx = _splat_at(x_v, nxt_s)
            qy = _splat_at(y_v, nxt_s)
            qz = _splat_at(z_v, nxt_s)
            return qx, qy, qz, pending

        lax.fori_loop(1, _M, _outer, (qx, qy, qz, zeros))
        pltpu.sync_copy(idx_v, out_hbm.at[wid])


@functools.partial(
    pl.kernel,
    mesh=plsc.VectorSubcoreMesh(core_axis_name="c", subcore_axis_name="s"),
    out_type=jax.ShapeDtypeStruct((_B, _M), jnp.int32),
    scratch_types=[
        pltpu.VMEM((_N + _L,), jnp.float32),
        pltpu.VMEM((_N + _L,), jnp.float32),
        pltpu.VMEM((_N + _L,), jnp.float32),
        pltpu.VMEM((_N,), jnp.float32),
        pltpu.VMEM((_M,), jnp.int32),
    ],
)
def _fps(pts_hbm, out_hbm, x_v, y_v, z_v, dists_v, idx_v):
    _fps_body(pts_hbm, out_hbm, x_v, y_v, z_v, dists_v, idx_v)


def kernel(pt_coordinates):
    return _fps(pt_coordinates.reshape(-1))


# speedup vs baseline: 2.5566x; 2.5193x over previous
import functools
import jax
import jax.numpy as jnp
from jax import lax
from jax.experimental import pallas as pl
from jax.experimental.pallas import tpu as pltpu
from jax.experimental.pallas import tpu_sc as plsc

_B, _C, _N = 8, 3, 16384
_M = 512
_L = 16
_CHUNKS = _N // _L
_UNROLL = 4


def _lane_permute(v, perm):
    dnums = lax.GatherDimensionNumbers(
        offset_dims=(), collapsed_slice_dims=(0,), start_index_map=(0,))
    return lax.gather(v, perm[:, None], dnums, (1,),
                      mode=lax.GatherScatterMode.PROMISE_IN_BOUNDS)


def _splat_at(ref, pos):
    return lax.broadcast(ref[pl.ds(pos, _L)][0], (_L,))


def _fps_body(pts_hbm, out_hbm, x_v, y_v, z_v, dists_v, idx_v):
    cid = lax.axis_index("c")
    sid = lax.axis_index("s")
    wid = sid * 2 + cid

    @pl.when(wid < _B)
    def _body():
        base_off = wid * (_C * _N)
        pltpu.sync_copy(pts_hbm.at[pl.ds(base_off, _N)], x_v.at[pl.ds(0, _N)])
        pltpu.sync_copy(pts_hbm.at[pl.ds(base_off + _N, _N)],
                        y_v.at[pl.ds(0, _N)])
        pltpu.sync_copy(pts_hbm.at[pl.ds(base_off + 2 * _N, _N)],
                        z_v.at[pl.ds(0, _N)])

        lanes = lax.iota(jnp.int32, _L)
        zeros = jnp.zeros((_L,), jnp.int32)
        intmax = jnp.full((_L,), 2147483647, jnp.int32)
        inf16 = jnp.full((_L,), jnp.inf, jnp.float32)

        def _init(j, carry):
            dists_v[pl.ds(j * _L, _L)] = inf16
            return carry

        lax.fori_loop(0, _CHUNKS, _init, 0)

        qx = _splat_at(x_v, 0)
        qy = _splat_at(y_v, 0)
        qz = _splat_at(z_v, 0)

        def _outer(i, carry):
            qx, qy, qz, pending = carry

            neg16 = jnp.full((_L,), -jnp.inf, jnp.float32)
            acc0 = (neg16, zeros) * _UNROLL

            @plsc.parallel_loop(0, _CHUNKS // _UNROLL, 1, unroll=2,
                                carry=acc0)
            def _chunk(j, st):
                jbase = j * (_L * _UNROLL)
                nds = []
                for u in range(_UNROLL):
                    base = jbase + u * _L
                    dx = x_v[pl.ds(base, _L)] - qx
                    dy = y_v[pl.ds(base, _L)] - qy
                    dz = z_v[pl.ds(base, _L)] - qz
                    d = dx * dx + dy * dy
                    d = d + dz * dz
                    nds.append(jnp.minimum(dists_v[pl.ds(base, _L)], d))
                for u in range(_UNROLL):
                    dists_v[pl.ds(jbase + u * _L, _L)] = nds[u]
                out = []
                for u in range(_UNROLL):
                    runmax, runidx = st[2 * u], st[2 * u + 1]
                    gt = nds[u] > runmax
                    out.append(jnp.where(gt, nds[u], runmax))
                    out.append(jnp.where(gt, lanes + (jbase + u * _L),
                                         runidx))
                return tuple(out)

            acc = _chunk
            runmax, runidx = acc[0], acc[1]
            for u in range(1, _UNROLL):
                bv, bi = acc[2 * u], acc[2 * u + 1]
                take = (bv > runmax) | ((bv == runmax) & (bi < runidx))
                runmax = jnp.where(take, bv, runmax)
                runidx = jnp.where(take, bi, runidx)

            for shift in (8, 4, 2, 1):
                perm = (lanes + shift) & (_L - 1)
                bv = _lane_permute(runmax, perm)
                bi = _lane_permute(runidx, perm)
                take = (bv > runmax) | ((bv == runmax) & (bi < runidx))
                runmax = jnp.where(take, bv, runmax)
                runidx = jnp.where(take, bi, runidx)
            nxt_s = runidx[0]

            pending = jnp.where(lanes == (i & (_L - 1)), runidx, pending)

            @pl.when((i & (_L - 1)) == _L - 1)
            def _flush():
                idx_v[pl.ds(i - (_L - 1), _L)] = pending

            qx = _splat_at(x_v, nxt_s)
            qy = _splat_at(y_v, nxt_s)
            qz = _splat_at(z_v, nxt_s)
            return qx, qy, qz, pending

        lax.fori_loop(1, _M, _outer, (qx, qy, qz, zeros))
        pltpu.sync_copy(idx_v, out_hbm.at[wid])


@functools.partial(
    pl.kernel,
    mesh=plsc.VectorSubcoreMesh(core_axis_name="c", subcore_axis_name="s"),
    out_type=jax.ShapeDtypeStruct((_B, _M), jnp.int32),
    scratch_types=[
        pltpu.VMEM((_N + _L,), jnp.float32),
        pltpu.VMEM((_N + _L,), jnp.float32),
        pltpu.VMEM((_N + _L,), jnp.float32),
        pltpu.VMEM((_N,), jnp.float32),
        pltpu.VMEM((_M,), jnp.int32),
    ],
)
def _fps(pts_hbm, out_hbm, x_v, y_v, z_v, dists_v, idx_v):
    _fps_body(pts_hbm, out_hbm, x_v, y_v, z_v, dists_v, idx_v)


def kernel(pt_coordinates):
    return _fps(pt_coordinates.reshape(-1))


# speedup vs baseline: 4.7808x; 1.8700x over previous
import functools
import jax
import jax.numpy as jnp
from jax import lax
from jax.experimental import pallas as pl
from jax.experimental.pallas import tpu as pltpu
from jax.experimental.pallas import tpu_sc as plsc

_B, _C, _N = 8, 3, 16384
_M = 512
_L = 16
_S = 4
_NS = _N // _S
_SCHUNKS = _NS // _L
_UNROLL = 4


def _lane_permute(v, perm):
    dnums = lax.GatherDimensionNumbers(
        offset_dims=(), collapsed_slice_dims=(0,), start_index_map=(0,))
    return lax.gather(v, perm[:, None], dnums, (1,),
                      mode=lax.GatherScatterMode.PROMISE_IN_BOUNDS)


def _splat_at(ref, pos):
    return lax.broadcast(ref[pl.ds(pos, _L)][0], (_L,))


def _fps_body(pts_hbm, out_hbm, xch_hbm, x_v, y_v, z_v, dists_v, idx_v,
              pub_v, gath_v):
    cid = lax.axis_index("c")
    sid = lax.axis_index("s")
    wid = cid * 16 + sid
    batch = cid * (_B // 2) + sid // _S
    shard = sid % _S
    gw0 = cid * 16 + (sid // _S) * _S
    sbase = shard * _NS

    base_off = batch * (_C * _N)
    pltpu.sync_copy(pts_hbm.at[pl.ds(base_off, _N)], x_v.at[pl.ds(0, _N)])
    pltpu.sync_copy(pts_hbm.at[pl.ds(base_off + _N, _N)],
                    y_v.at[pl.ds(0, _N)])
    pltpu.sync_copy(pts_hbm.at[pl.ds(base_off + 2 * _N, _N)],
                    z_v.at[pl.ds(0, _N)])

    lanes = lax.iota(jnp.int32, _L)
    zeros = jnp.zeros((_L,), jnp.int32)
    inf16 = jnp.full((_L,), jnp.inf, jnp.float32)

    def _init(j, carry):
        dists_v[pl.ds(j * _L, _L)] = inf16
        return carry

    lax.fori_loop(0, _SCHUNKS, _init, 0)

    qx = _splat_at(x_v, 0)
    qy = _splat_at(y_v, 0)
    qz = _splat_at(z_v, 0)

    def _outer(i, carry):
        qx, qy, qz, pending = carry

        neg16 = jnp.full((_L,), -jnp.inf, jnp.float32)
        acc0 = (neg16, zeros) * _UNROLL

        @plsc.parallel_loop(0, _SCHUNKS // _UNROLL, 1, unroll=2,
                            carry=acc0)
        def _chunk(j, st):
            jbase = j * (_L * _UNROLL)
            nds = []
            for u in range(_UNROLL):
                base = jbase + u * _L
                dx = x_v[pl.ds(sbase + base, _L)] - qx
                dy = y_v[pl.ds(sbase + base, _L)] - qy
                dz = z_v[pl.ds(sbase + base, _L)] - qz
                d = dx * dx + dy * dy
                d = d + dz * dz
                nds.append(jnp.minimum(dists_v[pl.ds(base, _L)], d))
            for u in range(_UNROLL):
                dists_v[pl.ds(jbase + u * _L, _L)] = nds[u]
            out = []
            for u in range(_UNROLL):
                runmax, runidx = st[2 * u], st[2 * u + 1]
                gt = nds[u] > runmax
                out.append(jnp.where(gt, nds[u], runmax))
                out.append(jnp.where(gt, lanes + (sbase + jbase + u * _L),
                                     runidx))
            return tuple(out)

        acc = _chunk
        runmax, runidx = acc[0], acc[1]
        for u in range(1, _UNROLL):
            bv, bi = acc[2 * u], acc[2 * u + 1]
            take = (bv > runmax) | ((bv == runmax) & (bi < runidx))
            runmax = jnp.where(take, bv, runmax)
            runidx = jnp.where(take, bi, runidx)

        for shift in (8, 4, 2, 1):
            perm = (lanes + shift) & (_L - 1)
            bv = _lane_permute(runmax, perm)
            bi = _lane_permute(runidx, perm)
            take = (bv > runmax) | ((bv == runmax) & (bi < runidx))
            runmax = jnp.where(take, bv, runmax)
            runidx = jnp.where(take, bi, runidx)

        pub_v[0] = runmax
        pub_v[1] = runidx.astype(jnp.float32)
        buf = i & 1
        pltpu.sync_copy(pub_v, xch_hbm.at[buf, wid])
        plsc.subcore_barrier()
        pltpu.sync_copy(xch_hbm.at[buf, pl.ds(gw0, _S)], gath_v)

        bestv = gath_v[0, 0]
        besti_f = gath_v[0, 1]
        for k in range(1, _S):
            vv = gath_v[k, 0]
            vi = gath_v[k, 1]
            take = (vv > bestv) | ((vv == bestv) & (vi < besti_f))
            bestv = jnp.where(take, vv, bestv)
            besti_f = jnp.where(take, vi, besti_f)
        besti = besti_f.astype(jnp.int32)
        nxt_s = besti[0]

        pending = jnp.where(lanes == (i & (_L - 1)), besti, pending)

        @pl.when((i & (_L - 1)) == _L - 1)
        def _flush():
            idx_v[pl.ds(i - (_L - 1), _L)] = pending

        qx = _splat_at(x_v, nxt_s)
        qy = _splat_at(y_v, nxt_s)
        qz = _splat_at(z_v, nxt_s)
        return qx, qy, qz, pending

    lax.fori_loop(1, _M, _outer, (qx, qy, qz, zeros))

    @pl.when(shard == 0)
    def _writeout():
        pltpu.sync_copy(idx_v, out_hbm.at[batch])


@functools.partial(
    pl.kernel,
    mesh=plsc.VectorSubcoreMesh(core_axis_name="c", subcore_axis_name="s"),
    out_type=[
        jax.ShapeDtypeStruct((_B, _M), jnp.int32),
        jax.ShapeDtypeStruct((2, 32, 2, _L), jnp.float32),
    ],
    scratch_types=[
        pltpu.VMEM((_N + _L,), jnp.float32),
        pltpu.VMEM((_N + _L,), jnp.float32),
        pltpu.VMEM((_N + _L,), jnp.float32),
        pltpu.VMEM((_NS,), jnp.float32),
        pltpu.VMEM((_M,), jnp.int32),
        pltpu.VMEM((2, _L), jnp.float32),
        pltpu.VMEM((_S, 2, _L), jnp.float32),
    ],
)
def _fps(pts_hbm, out_hbm, xch_hbm, x_v, y_v, z_v, dists_v, idx_v, pub_v,
         gath_v):
    _fps_body(pts_hbm, out_hbm, xch_hbm, x_v, y_v, z_v, dists_v, idx_v,
              pub_v, gath_v)


def kernel(pt_coordinates):
    idxs, _ = _fps(pt_coordinates.reshape(-1))
    return idxs
